# baseline (speedup 1.0000x reference)
import functools
import math

import jax
import jax.numpy as jnp
from jax import lax
from jax.experimental import pallas as pl
from jax.experimental.pallas import tpu as pltpu
from jax.experimental.pallas import tpu_sc as plsc

SCALE = 30.0
MARGIN = 0.5
COS_M = math.cos(MARGIN)
SIN_M = math.sin(MARGIN)
TH = math.cos(math.pi - MARGIN)
MM = math.sin(math.pi - MARGIN) * MARGIN

B = 1024
C = 100000


_info = plsc.get_sparse_core_info()
_NC, _NS = _info.num_cores, _info.num_subcores
_NW = _NC * _NS
_BPW = B // _NW


def _sc_gather_body(cos_flat, lab, out, lab_v, idx_v, val_v, sem):
    wid = lax.axis_index("s") * _NC + lax.axis_index("c")
    base = wid * _BPW
    pltpu.sync_copy(lab.at[pl.ds(base, _BPW)], lab_v)
    for s in range(_BPW // 16):
        lab16 = lab_v[pl.ds(s * 16, 16)]
        rows = (base + s * 16) + lax.iota(jnp.int32, 16)
        idx_v[pl.ds(s * 16, 16)] = rows * jnp.int32(C) + lab16
    pltpu.async_copy(cos_flat.at[idx_v], val_v, sem).wait()
    pltpu.sync_copy(val_v, out.at[pl.ds(base, _BPW)])


_sc_gather = functools.partial(
    pl.kernel,
    out_type=jax.ShapeDtypeStruct((B,), jnp.float32),
    mesh=plsc.VectorSubcoreMesh(core_axis_name="c", subcore_axis_name="s"),
    scratch_types=[
        pltpu.VMEM((_BPW,), jnp.int32),
        pltpu.VMEM((_BPW,), jnp.int32),
        pltpu.VMEM((_BPW,), jnp.float32),
        pltpu.SemaphoreType.DMA,
    ],
)(_sc_gather_body)


def _phi_body(c_ref, phi_ref):
    c = c_ref[...]
    sine = jnp.sqrt(1.0 - c * c)
    phi = c * COS_M - sine * SIN_M
    phi = jnp.where(c > TH, phi, c - MM)
    phi_ref[...] = phi * SCALE


def _phi_call(c_lab):
    out = pl.pallas_call(
        _phi_body,
        out_shape=jax.ShapeDtypeStruct((8, B // 8), jnp.float32),
    )(c_lab.reshape(8, B // 8))
    return out.reshape(B)


BR = 8
NSTEPS = B // BR
NBUF = 12
LA = 6
C_PAD = ((C + 127) // 128) * 128


def _stream_body(lab_smem, phi_smem, cos_hbm, out_hbm, buf, in_sems, out_sems):
    s = pl.program_id(0)
    slot = lax.rem(s, NBUF)

    def in_copy(step, k):
        return pltpu.make_async_copy(
            cos_hbm.at[pl.ds(step * BR, BR), :], buf.at[k], in_sems.at[k])

    def out_copy(step, k):
        return pltpu.make_async_copy(
            buf.at[k], out_hbm.at[pl.ds(step * BR, BR), :], out_sems.at[k])

    @pl.when(s == 0)
    def _():
        for k in range(LA):
            in_copy(k, k).start()

    @pl.when(s + LA < NSTEPS)
    def _():
        slot2 = lax.rem(s + LA, NBUF)

        @pl.when(s + LA >= NBUF)
        def _():
            out_copy(s + LA - NBUF, slot2).wait()

        in_copy(s + LA, slot2).start()

    in_copy(s, slot).wait()

    buf[slot] = buf[slot] * SCALE
    lane = lax.broadcasted_iota(jnp.int32, (1, 128), 1)
    for k in range(BR):
        r = s * BR + k
        col = lab_smem[r]
        base = pl.multiple_of((col // 128) * 128, 128)
        seg = buf[slot, pl.ds(k, 1), pl.ds(base, 128)]
        buf[slot, pl.ds(k, 1), pl.ds(base, 128)] = jnp.where(
            lane == (col - base), phi_smem[r], seg)

    out_copy(s, slot).start()

    @pl.when(s == NSTEPS - 1)
    def _():
        for k in range(NBUF):
            step = NSTEPS - 1 - k
            out_copy(step, lax.rem(step, NBUF)).wait()


def kernel(cosine_theta_logits, label):
    lab32 = label.astype(jnp.int32)
    c_lab = _sc_gather(cosine_theta_logits.reshape(-1), lab32)
    phi_scaled = _phi_call(c_lab)
    out = pl.pallas_call(
        _stream_body,
        grid=(NSTEPS,),
        in_specs=[
            pl.BlockSpec(memory_space=pltpu.SMEM),
            pl.BlockSpec(memory_space=pltpu.SMEM),
            pl.BlockSpec(memory_space=pl.ANY),
        ],
        out_specs=pl.BlockSpec(memory_space=pl.ANY),
        out_shape=jax.ShapeDtypeStruct((B, C), jnp.float32),
        scratch_shapes=[
            pltpu.VMEM((NBUF, BR, C), jnp.float32),
            pltpu.SemaphoreType.DMA((NBUF,)),
            pltpu.SemaphoreType.DMA((NBUF,)),
        ],
        compiler_params=pltpu.CompilerParams(
            dimension_semantics=("arbitrary",),
        ),
    )(lab32, phi_scaled, cosine_theta_logits)
    return out


# speedup vs baseline: 1.6149x; 1.6149x over previous
import math

import jax
import jax.numpy as jnp
from jax import lax
from jax.experimental import pallas as pl
from jax.experimental.pallas import tpu as pltpu

SCALE = 30.0
MARGIN = 0.5
COS_M = math.cos(MARGIN)
SIN_M = math.sin(MARGIN)
TH = math.cos(math.pi - MARGIN)
MM = math.sin(math.pi - MARGIN) * MARGIN

B = 1024
C = 100000

BR = 8
NSTEPS = B // BR
NBUF = 12
LA = 6


def _stream_body(lab_smem, cos_hbm, out_hbm, buf, in_sems, out_sems):
    s = pl.program_id(0)
    slot = lax.rem(s, NBUF)

    def in_copy(step, k):
        return pltpu.make_async_copy(
            cos_hbm.at[pl.ds(step * BR, BR), :], buf.at[k], in_sems.at[k])

    def out_copy(step, k):
        return pltpu.make_async_copy(
            buf.at[k], out_hbm.at[pl.ds(step * BR, BR), :], out_sems.at[k])

    @pl.when(s == 0)
    def _():
        for k in range(LA):
            in_copy(k, k).start()

    @pl.when(s + LA < NSTEPS)
    def _():
        slot2 = lax.rem(s + LA, NBUF)

        @pl.when(s + LA >= NBUF)
        def _():
            out_copy(s + LA - NBUF, slot2).wait()

        in_copy(s + LA, slot2).start()

    in_copy(s, slot).wait()

    lane = lax.broadcasted_iota(jnp.int32, (1, 128), 1)
    for k in range(BR):
        col = lab_smem[s * BR + k]
        base = pl.multiple_of((col // 128) * 128, 128)
        seg = buf[slot, pl.ds(k, 1), pl.ds(base, 128)]
        sine = jnp.sqrt(1.0 - seg * seg)
        phi = seg * COS_M - sine * SIN_M
        phi = jnp.where(seg > TH, phi, seg - MM)
        buf[slot, pl.ds(k, 1), pl.ds(base, 128)] = jnp.where(
            lane == (col - base), phi, seg)
    buf[slot] = buf[slot] * SCALE

    out_copy(s, slot).start()

    @pl.when(s == NSTEPS - 1)
    def _():
        for k in range(NBUF):
            step = NSTEPS - 1 - k
            out_copy(step, lax.rem(step, NBUF)).wait()


def kernel(cosine_theta_logits, label):
    lab32 = label.astype(jnp.int32)
    out = pl.pallas_call(
        _stream_body,
        grid=(NSTEPS,),
        in_specs=[
            pl.BlockSpec(memory_space=pltpu.SMEM),
            pl.BlockSpec(memory_space=pl.ANY),
        ],
        out_specs=pl.BlockSpec(memory_space=pl.ANY),
        out_shape=jax.ShapeDtypeStruct((B, C), jnp.float32),
        scratch_shapes=[
            pltpu.VMEM((NBUF, BR, C), jnp.float32),
            pltpu.SemaphoreType.DMA((NBUF,)),
            pltpu.SemaphoreType.DMA((NBUF,)),
        ],
        compiler_params=pltpu.CompilerParams(
            dimension_semantics=("arbitrary",),
        ),
    )(lab32, cosine_theta_logits)
    return out
